# Initial kernel scaffold; baseline (speedup 1.0000x reference)
import jax
import jax.numpy as jnp
from jax.experimental import pallas as pl


def kernel(dist):
    raise NotImplementedError("write your pallas kernel here")


# speedup vs baseline: 1.7548x; 1.7548x over previous
import functools

import jax
import jax.numpy as jnp
from jax import lax
from jax.experimental import pallas as pl
from jax.experimental.pallas import tpu as pltpu
from jax.experimental.pallas import tpu_sc as plsc

NROW = 128
NCOL = 100000
NSAMP = 256
L = 16
NB = NCOL // L
QB = (1600, 1600, 1600, 1450)
QSTRIDE = 1600
NBPAD = 4 * QSTRIDE
CB = 50
CHUNK_E = CB * L


def _iota16():
    return lax.iota(jnp.int32, 16)


def _bcast_i32(x):
    return x + jnp.zeros((16,), jnp.int32)


def _bcast_f32(x):
    return x + jnp.zeros((16,), jnp.float32)


def _phase1_body(dist_hbm, g16_hbm, buf, g16buf):
    cid = lax.axis_index("c")
    sid = lax.axis_index("s")
    wid = cid * 16 + sid
    g = wid // 4
    q = wid % 4
    nch = jnp.where(q == 3, 29, 32)
    blk0 = q * QSTRIDE
    iota = _iota16()
    rowbase = iota * CHUNK_E

    def chunk_body(ci, acc):
        e0 = (blk0 + ci * CB) * L
        pltpu.sync_copy(dist_hbm.at[pl.ds(g * 16, 16), pl.ds(e0, CHUNK_E)], buf)
        for b in range(CB):
            vals = [
                plsc.load_gather(buf, [iota, jnp.full((16,), b * L + j, jnp.int32)])
                for j in range(L)
            ]
            while len(vals) > 1:
                vals = [vals[i] + vals[i + 1] for i in range(0, len(vals), 2)]
            acc = acc + vals[0]
            col = _bcast_i32(ci * CB + b)
            plsc.store_scatter(g16buf, [iota, col], acc)
        return acc

    lax.fori_loop(0, nch, chunk_body, jnp.zeros((16,), jnp.float32))
    pltpu.sync_copy(g16buf, g16_hbm.at[pl.ds(g * 16, 16), pl.ds(blk0, QSTRIDE)])


def _phase2_body(dist2d_hbm, g16_hbm, u_hbm, out_hbm,
                 g16v, uv, gidx, thr, b16, rawbuf, outbuf, sem):
    cid = lax.axis_index("c")
    sid = lax.axis_index("s")
    wid = cid * 16 + sid
    r0 = wid * 4
    iota = _iota16()
    zeros_f = jnp.zeros((16,), jnp.float32)
    zeros_i = jnp.zeros((16,), jnp.int32)
    ones_i = jnp.ones((16,), jnp.int32)

    pltpu.sync_copy(g16_hbm.at[pl.ds(r0, 4), :], g16v)
    pltpu.sync_copy(u_hbm.at[pl.ds(r0 * NSAMP, 4 * NSAMP)], uv)

    def row_body(r, _):
        rfull = _bcast_i32(r)
        qt = [
            plsc.load_gather(
                g16v, [rfull, jnp.full((16,), q * QSTRIDE + QB[q] - 1, jnp.int32)])
            for q in range(4)
        ]
        qp1 = qt[0]
        qp2 = qp1 + qt[1]
        qp3 = qp2 + qt[2]
        tot = qp3 + qt[3]

        def search_body(jv, _):
            uvv = uv[pl.ds(r * NSAMP + jv * 16, 16)]
            t = uvv * tot
            qsel = (jnp.where(qp1 <= t, ones_i, zeros_i)
                    + jnp.where(qp2 <= t, ones_i, zeros_i)
                    + jnp.where(qp3 <= t, ones_i, zeros_i))
            qpsel = jnp.where(qsel == 0, zeros_f,
                              jnp.where(qsel == 1, qp1,
                                        jnp.where(qsel == 2, qp2, qp3)))
            tp = t - qpsel
            B = qsel * QSTRIDE
            n = jnp.where(qsel == 3, jnp.full((16,), QB[3], jnp.int32),
                          jnp.full((16,), QSTRIDE, jnp.int32))
            p = zeros_i
            for s in (1024, 512, 256, 128, 64, 32, 16, 8, 4, 2, 1):
                cand = p + s
                col = jnp.minimum(B + cand - 1, NBPAD - 1)
                val = plsc.load_gather(g16v, [rfull, col])
                ok = jnp.logical_and(cand <= n, val <= tp)
                p = jnp.where(ok, cand, p)
            basecol = jnp.minimum(jnp.maximum(B + p - 1, 0), NBPAD - 1)
            base = jnp.where(p > 0, plsc.load_gather(g16v, [rfull, basecol]),
                             zeros_f)
            blk = jnp.minimum(B + p, NB - 1)
            gi = (r0 + r) * NB + blk
            plsc.store_scatter(gidx, [_bcast_i32(jv // 8),
                                      iota + (jv % 8) * 16], gi)
            thr[pl.ds(jv * 16, 16)] = tp - base
            b16[pl.ds(jv * 16, 16)] = (B + p) * 16
            return 0

        lax.fori_loop(0, 16, search_body, 0)

        cp0 = pltpu.async_copy(dist2d_hbm.at[gidx.at[0]],
                               rawbuf.at[pl.ds(0, 128)], sem)
        cp1 = pltpu.async_copy(dist2d_hbm.at[gidx.at[1]],
                               rawbuf.at[pl.ds(128, 128)], sem)
        cp0.wait()
        cp1.wait()

        def refine_body(jv, _):
            tpv = thr[pl.ds(jv * 16, 16)]
            bv = b16[pl.ds(jv * 16, 16)]
            rowsel = iota + jv * 16
            run = zeros_f
            cnt = zeros_i
            for j in range(L):
                val = plsc.load_gather(
                    rawbuf, [rowsel, jnp.full((16,), j, jnp.int32)])
                run = run + val
                cnt = cnt + jnp.where(run <= tpv, ones_i, zeros_i)
            idx = jnp.minimum(jnp.maximum(bv + cnt, 0), NCOL - 1)
            outbuf[pl.ds(r * NSAMP + jv * 16, 16)] = idx
            return 0

        lax.fori_loop(0, 16, refine_body, 0)
        return 0

    lax.fori_loop(0, 4, row_body, 0)
    pltpu.sync_copy(outbuf, out_hbm.at[pl.ds(r0 * NSAMP, 4 * NSAMP)])


def kernel(dist):
    mesh = plsc.VectorSubcoreMesh(core_axis_name="c", subcore_axis_name="s")

    params = pltpu.CompilerParams(use_tc_tiling_on_sc=False,
                                  needs_layout_passes=False)

    phase1 = pl.kernel(
        _phase1_body,
        out_type=jax.ShapeDtypeStruct((NROW, NBPAD), jnp.float32),
        mesh=mesh,
        compiler_params=params,
        scratch_types=[
            pltpu.VMEM((16, CHUNK_E), jnp.float32),
            pltpu.VMEM((16, QSTRIDE), jnp.float32),
        ],
    )
    g16 = phase1(dist)

    ukey = jax.random.fold_in(jax.random.key(0), 1)
    u = jax.random.uniform(ukey, (NROW, NSAMP), dtype=jnp.float32)
    u_flat = u.reshape(NROW * NSAMP)
    dist2d = dist.reshape(NROW * NB, L)

    phase2 = pl.kernel(
        _phase2_body,
        out_type=jax.ShapeDtypeStruct((NROW * NSAMP,), jnp.int32),
        mesh=mesh,
        compiler_params=params,
        scratch_types=[
            pltpu.VMEM((4, NBPAD), jnp.float32),
            pltpu.VMEM((4 * NSAMP,), jnp.float32),
            pltpu.VMEM((2, 128), jnp.int32),
            pltpu.VMEM((NSAMP,), jnp.float32),
            pltpu.VMEM((NSAMP,), jnp.int32),
            pltpu.VMEM((NSAMP, L), jnp.float32),
            pltpu.VMEM((4 * NSAMP,), jnp.int32),
            pltpu.SemaphoreType.DMA,
        ],
    )
    return phase2(dist2d, g16, u_flat).reshape(NROW, NSAMP)


# speedup vs baseline: 2.6022x; 1.4829x over previous
import functools

import jax
import jax.numpy as jnp
from jax import lax
from jax.experimental import pallas as pl
from jax.experimental.pallas import tpu as pltpu
from jax.experimental.pallas import tpu_sc as plsc

NROW = 128
NCOL = 100000
NSAMP = 256
L = 16
NB = NCOL // L
QB = (1600, 1600, 1600, 1450)
QSTRIDE = 1600
NBPAD = 4 * QSTRIDE
CB = 25
CHUNK_E = CB * L


def _iota16():
    return lax.iota(jnp.int32, 16)


def _bcast_i32(x):
    return x + jnp.zeros((16,), jnp.int32)


def _bcast_f32(x):
    return x + jnp.zeros((16,), jnp.float32)


def _phase1_body(dist_hbm, g16_hbm, buf_a, buf_b, g16buf, sem_a, sem_b):
    cid = lax.axis_index("c")
    sid = lax.axis_index("s")
    wid = cid * 16 + sid
    g = wid // 4
    q = wid % 4
    npairs = jnp.where(q == 3, 29, 32)
    blk0 = q * QSTRIDE
    e_base = blk0 * L
    iota = _iota16()

    def src(ci):
        return dist_hbm.at[pl.ds(g * 16, 16),
                           pl.ds(e_base + ci * CHUNK_E, CHUNK_E)]

    def compute(buf, ci, acc):
        for b in range(CB):
            vals = [
                plsc.load_gather(buf, [iota, jnp.full((16,), b * L + j, jnp.int32)])
                for j in range(L)
            ]
            while len(vals) > 1:
                vals = [vals[i] + vals[i + 1] for i in range(0, len(vals), 2)]
            acc = acc + vals[0]
            plsc.store_scatter(g16buf, [iota, _bcast_i32(ci * CB + b)], acc)
        return acc

    pltpu.make_async_copy(src(0), buf_a, sem_a).start()

    def pair_body(pi, acc):
        ci = 2 * pi
        pltpu.make_async_copy(src(ci), buf_a, sem_a).wait()
        pltpu.make_async_copy(src(ci + 1), buf_b, sem_b).start()
        acc = compute(buf_a, ci, acc)
        pltpu.make_async_copy(src(ci + 1), buf_b, sem_b).wait()

        @pl.when(pi + 1 < npairs)
        def _prefetch():
            pltpu.make_async_copy(src(ci + 2), buf_a, sem_a).start()

        acc = compute(buf_b, ci + 1, acc)
        return acc

    lax.fori_loop(0, npairs, pair_body, jnp.zeros((16,), jnp.float32))
    pltpu.sync_copy(g16buf, g16_hbm.at[pl.ds(g * 16, 16), pl.ds(blk0, QSTRIDE)])


def _phase2_body(dist2d_hbm, g16_hbm, u_hbm, out_hbm,
                 g16v, uv, gidx, thr, b16, rawbuf, outbuf, sem):
    cid = lax.axis_index("c")
    sid = lax.axis_index("s")
    wid = cid * 16 + sid
    r0 = wid * 4
    iota = _iota16()
    zeros_f = jnp.zeros((16,), jnp.float32)
    zeros_i = jnp.zeros((16,), jnp.int32)
    ones_i = jnp.ones((16,), jnp.int32)

    pltpu.sync_copy(g16_hbm.at[pl.ds(r0, 4), :], g16v)
    pltpu.sync_copy(u_hbm.at[pl.ds(r0 * NSAMP, 4 * NSAMP)], uv)

    def row_body(r, _):
        rfull = _bcast_i32(r)
        qt = [
            plsc.load_gather(
                g16v, [rfull, jnp.full((16,), q * QSTRIDE + QB[q] - 1, jnp.int32)])
            for q in range(4)
        ]
        qp1 = qt[0]
        qp2 = qp1 + qt[1]
        qp3 = qp2 + qt[2]
        tot = qp3 + qt[3]

        def search_body(jv, _):
            uvv = uv[pl.ds(r * NSAMP + jv * 16, 16)]
            t = uvv * tot
            qsel = (jnp.where(qp1 <= t, ones_i, zeros_i)
                    + jnp.where(qp2 <= t, ones_i, zeros_i)
                    + jnp.where(qp3 <= t, ones_i, zeros_i))
            qpsel = jnp.where(qsel == 0, zeros_f,
                              jnp.where(qsel == 1, qp1,
                                        jnp.where(qsel == 2, qp2, qp3)))
            tp = t - qpsel
            B = qsel * QSTRIDE
            n = jnp.where(qsel == 3, jnp.full((16,), QB[3], jnp.int32),
                          jnp.full((16,), QSTRIDE, jnp.int32))
            p = zeros_i
            for s in (1024, 512, 256, 128, 64, 32, 16, 8, 4, 2, 1):
                cand = p + s
                col = jnp.minimum(B + cand - 1, NBPAD - 1)
                val = plsc.load_gather(g16v, [rfull, col])
                ok = jnp.logical_and(cand <= n, val <= tp)
                p = jnp.where(ok, cand, p)
            basecol = jnp.minimum(jnp.maximum(B + p - 1, 0), NBPAD - 1)
            base = jnp.where(p > 0, plsc.load_gather(g16v, [rfull, basecol]),
                             zeros_f)
            blk = jnp.minimum(B + p, NB - 1)
            gi = (r0 + r) * NB + blk
            plsc.store_scatter(gidx, [_bcast_i32(jv // 8),
                                      iota + (jv % 8) * 16], gi)
            thr[pl.ds(jv * 16, 16)] = tp - base
            b16[pl.ds(jv * 16, 16)] = (B + p) * 16
            return 0

        lax.fori_loop(0, 16, search_body, 0)

        cp0 = pltpu.async_copy(dist2d_hbm.at[gidx.at[0]],
                               rawbuf.at[pl.ds(0, 128)], sem)
        cp1 = pltpu.async_copy(dist2d_hbm.at[gidx.at[1]],
                               rawbuf.at[pl.ds(128, 128)], sem)
        cp0.wait()
        cp1.wait()

        def refine_body(jv, _):
            tpv = thr[pl.ds(jv * 16, 16)]
            bv = b16[pl.ds(jv * 16, 16)]
            rowsel = iota + jv * 16
            run = zeros_f
            cnt = zeros_i
            for j in range(L):
                val = plsc.load_gather(
                    rawbuf, [rowsel, jnp.full((16,), j, jnp.int32)])
                run = run + val
                cnt = cnt + jnp.where(run <= tpv, ones_i, zeros_i)
            idx = jnp.minimum(jnp.maximum(bv + cnt, 0), NCOL - 1)
            outbuf[pl.ds(r * NSAMP + jv * 16, 16)] = idx
            return 0

        lax.fori_loop(0, 16, refine_body, 0)
        return 0

    lax.fori_loop(0, 4, row_body, 0)
    pltpu.sync_copy(outbuf, out_hbm.at[pl.ds(r0 * NSAMP, 4 * NSAMP)])


def kernel(dist):
    mesh = plsc.VectorSubcoreMesh(core_axis_name="c", subcore_axis_name="s")

    params = pltpu.CompilerParams(use_tc_tiling_on_sc=False,
                                  needs_layout_passes=False)

    phase1 = pl.kernel(
        _phase1_body,
        out_type=jax.ShapeDtypeStruct((NROW, NBPAD), jnp.float32),
        mesh=mesh,
        compiler_params=params,
        scratch_types=[
            pltpu.VMEM((16, CHUNK_E), jnp.float32),
            pltpu.VMEM((16, CHUNK_E), jnp.float32),
            pltpu.VMEM((16, QSTRIDE), jnp.float32),
            pltpu.SemaphoreType.DMA,
            pltpu.SemaphoreType.DMA,
        ],
    )
    g16 = phase1(dist)

    ukey = jax.random.fold_in(jax.random.key(0), 1)
    u = jax.random.uniform(ukey, (NROW, NSAMP), dtype=jnp.float32)
    u_flat = u.reshape(NROW * NSAMP)
    dist2d = dist.reshape(NROW * NB, L)

    phase2 = pl.kernel(
        _phase2_body,
        out_type=jax.ShapeDtypeStruct((NROW * NSAMP,), jnp.int32),
        mesh=mesh,
        compiler_params=params,
        scratch_types=[
            pltpu.VMEM((4, NBPAD), jnp.float32),
            pltpu.VMEM((4 * NSAMP,), jnp.float32),
            pltpu.VMEM((2, 128), jnp.int32),
            pltpu.VMEM((NSAMP,), jnp.float32),
            pltpu.VMEM((NSAMP,), jnp.int32),
            pltpu.VMEM((NSAMP, L), jnp.float32),
            pltpu.VMEM((4 * NSAMP,), jnp.int32),
            pltpu.SemaphoreType.DMA,
        ],
    )
    return phase2(dist2d, g16, u_flat).reshape(NROW, NSAMP)


# speedup vs baseline: 2.8384x; 1.0907x over previous
import jax
import jax.numpy as jnp
from jax import lax
from jax.experimental import pallas as pl
from jax.experimental.pallas import tpu as pltpu
from jax.experimental.pallas import tpu_sc as plsc

NROW = 128
NCOL = 100000
NSAMP = 256
L = 16
NB = NCOL // L
SEG = 3120
SEGB = SEG // L
HALF = SEG * L
HBLK = HALF // L
TAIL = NCOL - 2 * HALF
TAILB = TAIL // L
G16W = 6256


def _iota16():
    return lax.iota(jnp.int32, 16)


def _bcast_i32(x):
    return x + jnp.zeros((16,), jnp.int32)


def _bcast_f32(x):
    return x + jnp.zeros((16,), jnp.float32)


def _body(dist_hbm, u_hbm, out_hbm,
          buf_a, buf_b, tailbuf, g16, uv, outbuf, s16,
          sem_a, sem_b, sem_t):
    cid = lax.axis_index("c")
    sid = lax.axis_index("s")
    wid = cid * 16 + sid
    iota = _iota16()
    seg_iota = iota * SEG
    blk_iota = iota * SEGB
    zeros_f = jnp.zeros((16,), jnp.float32)
    zeros_i = jnp.zeros((16,), jnp.int32)
    ones_i = jnp.ones((16,), jnp.int32)

    pltpu.sync_copy(u_hbm.at[pl.ds(wid * 4 * NSAMP, 4 * NSAMP)], uv)

    def compute_half(buf, blkoff):
        def blk_body(b, acc):
            e0 = b * L
            vals = [
                plsc.load_gather(buf, [seg_iota + (e0 + j)]) for j in range(L)
            ]
            while len(vals) > 1:
                vals = [vals[i] + vals[i + 1] for i in range(0, len(vals), 2)]
            acc = acc + vals[0]
            plsc.store_scatter(g16, [blk_iota + (blkoff + b)], acc)
            return acc

        return lax.fori_loop(0, SEGB, blk_body, zeros_f)

    def fixup_half(blkoff, base):
        def fix_body(b, _):
            idx = blk_iota + (blkoff + b)
            v = plsc.load_gather(g16, [idx])
            plsc.store_scatter(g16, [idx], v + base)
            return 0

        lax.fori_loop(0, SEGB, fix_body, 0)

    def row_body(rl, _):
        row = wid * 4 + rl
        e_row = row * NCOL
        cp_a = pltpu.make_async_copy(
            dist_hbm.at[pl.ds(e_row, HALF)], buf_a, sem_a)
        cp_b = pltpu.make_async_copy(
            dist_hbm.at[pl.ds(e_row + HALF, HALF)], buf_b, sem_b)
        cp_t = pltpu.make_async_copy(
            dist_hbm.at[pl.ds(e_row + 2 * HALF, TAIL)], tailbuf, sem_t)
        cp_a.start()
        cp_b.start()
        cp_t.start()

        cp_a.wait()
        acc_a = compute_half(buf_a, 0)
        cp_b.wait()
        acc_b = compute_half(buf_b, HBLK)

        cum_a = plsc.cumsum(acc_a)
        base_a = cum_a - acc_a
        s16[...] = cum_a
        tot_a = plsc.load_gather(s16, [jnp.full((16,), 15, jnp.int32)])
        cum_b = plsc.cumsum(acc_b)
        base_b = cum_b - acc_b + tot_a
        s16[...] = cum_b + tot_a
        tot_ab = plsc.load_gather(s16, [jnp.full((16,), 15, jnp.int32)])
        fixup_half(0, base_a)
        fixup_half(HBLK, base_b)

        cp_t.wait()
        lane0 = iota == 0
        tcum = tot_ab
        for t in range(TAILB):
            tcum = tcum + jnp.sum(tailbuf[pl.ds(t * L, L)])
            plsc.store_scatter(g16, [_bcast_i32(2 * HBLK + t)], tcum,
                               mask=lane0)

        tot = plsc.load_gather(g16, [jnp.full((16,), NB - 1, jnp.int32)])

        def samp_body(jv, _):
            t = uv[pl.ds(rl * NSAMP + jv * 16, 16)] * tot
            p = zeros_i
            for s in (4096, 2048, 1024, 512, 256, 128, 64, 32, 16, 8, 4, 2, 1):
                cand = p + s
                col = jnp.minimum(cand - 1, G16W - 1)
                val = plsc.load_gather(g16, [col])
                ok = jnp.logical_and(cand <= NB, val <= t)
                p = jnp.where(ok, cand, p)
            base = jnp.where(
                p > 0,
                plsc.load_gather(g16, [jnp.maximum(p - 1, 0)]), zeros_f)
            thr = t - base
            pc = jnp.minimum(p, NB - 1)
            e = pc * L
            run = zeros_f
            cnt = zeros_i
            for j in range(L):
                ej = e + j
                va = plsc.load_gather(
                    buf_a, [jnp.minimum(ej, HALF - 1)])
                vb = plsc.load_gather(
                    buf_b, [jnp.clip(ej - HALF, 0, HALF - 1)])
                vt = plsc.load_gather(
                    tailbuf, [jnp.clip(ej - 2 * HALF, 0, TAIL - 1)])
                val = jnp.where(ej < HALF, va,
                                jnp.where(ej < 2 * HALF, vb, vt))
                run = run + val
                cnt = cnt + jnp.where(run <= thr, ones_i, zeros_i)
            idx = jnp.minimum(jnp.maximum(p * L + cnt, 0), NCOL - 1)
            outbuf[pl.ds(rl * NSAMP + jv * 16, 16)] = idx
            return 0

        lax.fori_loop(0, 16, samp_body, 0)
        return 0

    lax.fori_loop(0, 4, row_body, 0)
    pltpu.sync_copy(outbuf, out_hbm.at[pl.ds(wid * 4 * NSAMP, 4 * NSAMP)])


def kernel(dist):
    mesh = plsc.VectorSubcoreMesh(core_axis_name="c", subcore_axis_name="s")
    params = pltpu.CompilerParams(use_tc_tiling_on_sc=False,
                                  needs_layout_passes=False)

    ukey = jax.random.fold_in(jax.random.key(0), 1)
    u = jax.random.uniform(ukey, (NROW, NSAMP), dtype=jnp.float32)

    run = pl.kernel(
        _body,
        out_type=jax.ShapeDtypeStruct((NROW * NSAMP,), jnp.int32),
        mesh=mesh,
        compiler_params=params,
        scratch_types=[
            pltpu.VMEM((HALF,), jnp.float32),
            pltpu.VMEM((HALF,), jnp.float32),
            pltpu.VMEM((TAIL,), jnp.float32),
            pltpu.VMEM((G16W,), jnp.float32),
            pltpu.VMEM((4 * NSAMP,), jnp.float32),
            pltpu.VMEM((4 * NSAMP,), jnp.int32),
            pltpu.VMEM((16,), jnp.float32),
            pltpu.SemaphoreType.DMA,
            pltpu.SemaphoreType.DMA,
            pltpu.SemaphoreType.DMA,
        ],
    )
    return run(dist.reshape(NROW * NCOL), u.reshape(NROW * NSAMP)).reshape(
        NROW, NSAMP)
